# Initial kernel scaffold; baseline (speedup 1.0000x reference)
import jax
import jax.numpy as jnp
from jax.experimental import pallas as pl


def kernel(x, edge_index, W1, b1, W2, b2, W3, b3, We, be):
    raise NotImplementedError("write your pallas kernel here")


# speedup vs baseline: 13.1512x; 13.1512x over previous
import functools

import jax
import jax.numpy as jnp
from jax import lax
from jax.experimental import pallas as pl
from jax.experimental.pallas import tpu as pltpu
from jax.experimental.pallas import tpu_sc as plsc

N = 100000
E = 1600000
IN_DIM = 128
H = 32
EMB = 16

NC = 2
NS = 16
NW = NC * NS
HALF = N // NC

EC = E // NS
EB = 2000
NVEC = EB // 16
NEBLK = EC // EB

GB = 512
NSUB = GB // 128
MAXBLK = (EC + GB - 1) // GB + 1
CAP = (MAXBLK + 1) * GB

ACC_ROWS = HALF + NS
ZCH = ACC_ROWS // NS
DCH = HALF // NS
DSTEP = 625

R = 2000
NROWBLK = N // R
HB = N // NC // R


def _sc_mesh():
    return plsc.VectorSubcoreMesh(core_axis_name="c", subcore_axis_name="s")


def _pre_body(src_hbm, dst_hbm, deg_part, src_p, dst_p, counts,
              sel, st_s, st_d, cvm, sem):
    c = lax.axis_index("c")
    s = lax.axis_index("s")
    wid = c * NS + s
    lo = c * HALF
    iota = lax.iota(jnp.int32, 16)
    ones = jnp.ones((16,), jnp.int32)
    zeros16 = jnp.zeros((16,), jnp.int32)

    def z_body(i, _):
        sel[pl.ds(i * 16, 16)] = zeros16
        return 0
    lax.fori_loop(0, N // 16, z_body, 0)

    def hblk(b, _):
        base = s * EC + b * EB
        pltpu.sync_copy(src_hbm.at[pl.ds(base, EB)], st_s)
        pltpu.sync_copy(dst_hbm.at[pl.ds(base, EB)], st_d)

        def hvec(i, _):
            sv = st_s[pl.ds(i * 16, 16)]
            dv = st_d[pl.ds(i * 16, 16)]
            mo = (sv >= lo) & (sv < lo + HALF)
            mi = (dv >= lo) & (dv < lo + HALF)
            plsc.addupdate_scatter(sel, [sv - lo], ones, mask=mo)
            plsc.addupdate_scatter(sel, [dv - lo + HALF], ones, mask=mi)
            return 0
        lax.fori_loop(0, NVEC, hvec, 0)
        return 0
    lax.fori_loop(0, NEBLK, hblk, 0)
    pltpu.sync_copy(sel, deg_part.at[wid])

    for which in range(2):
        def pblk(b, cur):
            base = s * EC + b * EB
            pltpu.sync_copy(src_hbm.at[pl.ds(base, EB)], st_s)
            pltpu.sync_copy(dst_hbm.at[pl.ds(base, EB)], st_d)

            def pvec(i, cur):
                sv = st_s[pl.ds(i * 16, 16)]
                dv = st_d[pl.ds(i * 16, 16)]
                m = (dv >= lo) & (dv < lo + HALF)
                mi32 = jnp.where(m, 1, 0).astype(jnp.int32)
                pos = cur + plsc.cumsum(mi32) - 1
                val = sv if which == 0 else dv - lo
                plsc.store_scatter(sel, [pos], val, mask=m)
                return cur + jnp.sum(mi32)
            return lax.fori_loop(0, NVEC, pvec, cur)
        cnt = lax.fori_loop(0, NEBLK, pblk, jnp.int32(0))

        for j in range(GB // 16):
            if which == 0:
                padv = wid * 3000 + j * 16 + iota
            else:
                padv = jnp.full((16,), HALF, jnp.int32) + s
            plsc.store_scatter(sel, [cnt + j * 16 + iota], padv)

        if which == 0:
            pltpu.sync_copy(sel, src_p.at[wid])
            cvm[...] = jnp.broadcast_to(cnt, (16,))
            pltpu.sync_copy(cvm, counts.at[wid])
        else:
            pltpu.sync_copy(sel, dst_p.at[wid])


@functools.partial(jax.jit, donate_argnums=())
def _pre(src, dst):
    f = pl.kernel(
        _pre_body,
        out_type=(
            jax.ShapeDtypeStruct((NW, CAP), jnp.int32),
            jax.ShapeDtypeStruct((NW, CAP), jnp.int32),
            jax.ShapeDtypeStruct((NW, CAP), jnp.int32),
            jax.ShapeDtypeStruct((NW, 16), jnp.int32),
        ),
        mesh=_sc_mesh(),
        scratch_types=[
            pltpu.VMEM((CAP,), jnp.int32),
            pltpu.VMEM((EB,), jnp.int32),
            pltpu.VMEM((EB,), jnp.int32),
            pltpu.VMEM((16,), jnp.int32),
            pltpu.SemaphoreType.DMA,
        ],
        compiler_params=pltpu.CompilerParams(needs_layout_passes=False, use_tc_tiling_on_sc=False),
    )
    return f(src, dst)


def _layer_body(h_hbm, srcp, dstp, counts, agg, acc, sidx, didx, msg, cvm, sem):
    c = lax.axis_index("c")
    s = lax.axis_index("s")
    wid = c * NS + s

    zf = jnp.zeros((16,), jnp.float32)

    def zb(i, _):
        msg[i, pl.ds(0, 16)] = zf
        msg[i, pl.ds(16, 16)] = zf
        return 0
    lax.fori_loop(0, GB, zb, 0)
    zbase = s * ZCH
    for k in range(6):
        pltpu.sync_copy(msg, acc.at[pl.ds(zbase + k * GB, GB)])
    pltpu.sync_copy(msg.at[pl.ds(0, ZCH - 6 * GB)],
                    acc.at[pl.ds(zbase + 6 * GB, ZCH - 6 * GB)])
    plsc.subcore_barrier()

    pltpu.sync_copy(counts.at[wid], cvm)
    cnt = cvm[...][0]
    nblk = lax.div(cnt + (GB - 1), GB)

    def bb(b, _):
        pltpu.sync_copy(srcp.at[wid, b], sidx)
        pltpu.sync_copy(dstp.at[wid, b], didx)
        cps = [pltpu.async_copy(h_hbm.at[sidx.at[j]],
                                msg.at[pl.ds(j * 128, 128)], sem)
               for j in range(NSUB)]
        for cp in cps:
            cp.wait()
        for j in range(NSUB):
            pltpu.sync_copy(msg.at[pl.ds(j * 128, 128)],
                            acc.at[didx.at[j]], add=True)
        return 0
    lax.fori_loop(0, nblk, bb, 0)
    plsc.subcore_barrier()

    dbase = s * DCH
    for k in range(6):
        pltpu.sync_copy(acc.at[pl.ds(dbase + k * GB, GB)], msg)
        pltpu.sync_copy(msg, agg.at[pl.ds(c * HALF + dbase + k * GB, GB)])
    tail = DCH - 6 * GB
    pltpu.sync_copy(acc.at[pl.ds(dbase + 6 * GB, tail)], msg.at[pl.ds(0, tail)])
    pltpu.sync_copy(msg.at[pl.ds(0, tail)],
                    agg.at[pl.ds(c * HALF + dbase + 6 * GB, tail)])


@jax.jit
def _layer(h, srcp4, dstp4, counts):
    f = pl.kernel(
        _layer_body,
        out_type=jax.ShapeDtypeStruct((N, H), jnp.float32),
        mesh=_sc_mesh(),
        scratch_types=[
            pltpu.VMEM_SHARED((ACC_ROWS, H), jnp.float32),
            pltpu.VMEM((NSUB, 128), jnp.int32),
            pltpu.VMEM((NSUB, 128), jnp.int32),
            pltpu.VMEM((GB, H), jnp.float32),
            pltpu.VMEM((16,), jnp.int32),
            pltpu.SemaphoreType.DMA,
        ],
        compiler_params=pltpu.CompilerParams(needs_layout_passes=False, use_tc_tiling_on_sc=False),
    )
    return f(h, srcp4, dstp4, counts)


def _tc1_body(x_ref, dgo_ref, dgi_ref, w1_ref, hs_ref, ns_ref, nd_ref):
    do = jnp.sum(dgo_ref[:, 0, 0, :].astype(jnp.float32), axis=0)
    di = jnp.sum(dgi_ref[:, 0, 0, :].astype(jnp.float32), axis=0)
    ns = lax.rsqrt(jnp.maximum(do, 1.0))
    nd = lax.rsqrt(jnp.maximum(di, 1.0))
    y = jnp.dot(x_ref[...], w1_ref[...], preferred_element_type=jnp.float32)
    hs_ref[...] = y * ns[:, None]
    ns_ref[...] = ns[None, None, :]
    nd_ref[...] = nd[None, None, :]


@jax.jit
def _tc1(x, deg3, W1):
    return pl.pallas_call(
        _tc1_body,
        grid=(NROWBLK,),
        in_specs=[
            pl.BlockSpec((R, IN_DIM), lambda i: (i, 0)),
            pl.BlockSpec((NS, 1, 1, R),
                         lambda i: (lax.div(i, HB), lax.rem(i, HB), 0, 0)),
            pl.BlockSpec((NS, 1, 1, R),
                         lambda i: (lax.div(i, HB), HB + lax.rem(i, HB), 0, 0)),
            pl.BlockSpec((IN_DIM, H), lambda i: (0, 0)),
        ],
        out_specs=[
            pl.BlockSpec((R, H), lambda i: (i, 0)),
            pl.BlockSpec((1, 1, R), lambda i: (i, 0, 0)),
            pl.BlockSpec((1, 1, R), lambda i: (i, 0, 0)),
        ],
        out_shape=[
            jax.ShapeDtypeStruct((N, H), jnp.float32),
            jax.ShapeDtypeStruct((NROWBLK, 1, R), jnp.float32),
            jax.ShapeDtypeStruct((NROWBLK, 1, R), jnp.float32),
        ],
    )(x, deg3, deg3, W1)


def _tcmid_body(agg_ref, nd_ref, ns_ref, w_ref, b_ref, hs_ref):
    t = jnp.maximum(agg_ref[...] * nd_ref[0, 0][:, None] + b_ref[...], 0.0)
    y = jnp.dot(t, w_ref[...], preferred_element_type=jnp.float32)
    hs_ref[...] = y * ns_ref[0, 0][:, None]


@jax.jit
def _tcmid(agg, nd, ns, W, b):
    return pl.pallas_call(
        _tcmid_body,
        grid=(NROWBLK,),
        in_specs=[
            pl.BlockSpec((R, H), lambda i: (i, 0)),
            pl.BlockSpec((1, 1, R), lambda i: (i, 0, 0)),
            pl.BlockSpec((1, 1, R), lambda i: (i, 0, 0)),
            pl.BlockSpec((H, H), lambda i: (0, 0)),
            pl.BlockSpec((1, H), lambda i: (0, 0)),
        ],
        out_specs=pl.BlockSpec((R, H), lambda i: (i, 0)),
        out_shape=jax.ShapeDtypeStruct((N, H), jnp.float32),
    )(agg, nd, ns, W, b.reshape(1, H))


def _tcfin_body(agg_ref, nd_ref, b3_ref, we_ref, be_ref, out_ref, acc_ref):
    i = pl.program_id(0)

    @pl.when(i == 0)
    def _():
        acc_ref[...] = jnp.zeros_like(acc_ref)

    t = jnp.maximum(agg_ref[...] * nd_ref[0, 0][:, None] + b3_ref[...], 0.0)
    acc_ref[...] += jnp.sum(t, axis=0, keepdims=True)

    @pl.when(i == NROWBLK - 1)
    def _():
        hg = acc_ref[...] * (1.0 / N)
        out_ref[...] = (jnp.dot(hg, we_ref[...],
                                preferred_element_type=jnp.float32)
                        + be_ref[...])


@jax.jit
def _tcfin(agg, nd, b3, We, be):
    return pl.pallas_call(
        _tcfin_body,
        grid=(NROWBLK,),
        in_specs=[
            pl.BlockSpec((R, H), lambda i: (i, 0)),
            pl.BlockSpec((1, 1, R), lambda i: (i, 0, 0)),
            pl.BlockSpec((1, H), lambda i: (0, 0)),
            pl.BlockSpec((H, EMB), lambda i: (0, 0)),
            pl.BlockSpec((1, EMB), lambda i: (0, 0)),
        ],
        out_specs=pl.BlockSpec((1, EMB), lambda i: (0, 0)),
        out_shape=jax.ShapeDtypeStruct((1, EMB), jnp.float32),
        scratch_shapes=[pltpu.VMEM((1, H), jnp.float32)],
    )(agg, nd, b3.reshape(1, H), We, be.reshape(1, EMB))


def kernel(x, edge_index, W1, b1, W2, b2, W3, b3, We, be):
    src = edge_index[0]
    dst = edge_index[1]
    deg_part, src_p, dst_p, counts = _pre(src, dst)
    srcp4 = src_p.reshape(NW, MAXBLK + 1, NSUB, 128)
    dstp4 = dst_p.reshape(NW, MAXBLK + 1, NSUB, 128)
    deg3 = deg_part[:, :N].reshape(NW, 2 * HB, 1, R)
    hs1, ns, nd = _tc1(x, deg3, W1)
    agg1 = _layer(hs1, srcp4, dstp4, counts)
    hs2 = _tcmid(agg1, nd, ns, W2, b1)
    agg2 = _layer(hs2, srcp4, dstp4, counts)
    hs3 = _tcmid(agg2, nd, ns, W3, b2)
    agg3 = _layer(hs3, srcp4, dstp4, counts)
    return _tcfin(agg3, nd, b3, We, be)
